# baseline (speedup 1.0000x reference)
import jax
import jax.numpy as jnp
from jax.experimental import pallas as pl

B = 16384
D = 128
BS = 2048

_DNT = (((1,), (1,)), ((), ()))


def _fused_mlp(xin_ref, w0a_ref, b0a_ref, w0b_ref, b0b_ref, w1p_ref, b1p_ref,
               w1q_ref, b1q_ref, o_ref):
    f32 = jnp.float32
    x0 = xin_ref[:, :D]
    x1 = xin_ref[:, D:2 * D]
    w0a, w0b, w1p, w1q = w0a_ref[0], w0b_ref[0], w1p_ref[0], w1q_ref[0]
    h0 = jnp.maximum(
        jax.lax.dot_general(x0, w0a, _DNT, preferred_element_type=f32)
        + b0a_ref[0:1, :], 0.0)
    x = jnp.maximum(
        jax.lax.dot_general(h0, w0b, _DNT, preferred_element_type=f32)
        + b0b_ref[0:1, :], 0.0)
    h1 = jnp.maximum(
        jax.lax.dot_general(x1, w1p, _DNT, preferred_element_type=f32)
        + b1p_ref[0:1, :], 0.0)
    o_ref[...] = (
        jax.lax.dot_general(x, w1q[:, :D], _DNT, preferred_element_type=f32)
        + jax.lax.dot_general(h1, w1q[:, D:], _DNT, preferred_element_type=f32)
        + b1q_ref[0:1, :])


def kernel(input_val, W0a, b0a, W0b, b0b, W1p, b1p, W1q, b1q):
    n_blocks = B // BS
    wspec = pl.BlockSpec((1, D, D), lambda i: (0, 0, 0))
    wspec2 = pl.BlockSpec((1, D, 2 * D), lambda i: (0, 0, 0))
    bspec = pl.BlockSpec((8, D), lambda i: (0, 0))
    out = pl.pallas_call(
        _fused_mlp,
        grid=(n_blocks,),
        in_specs=[
            pl.BlockSpec((BS, 258), lambda i: (i, 0)),
            wspec, bspec, wspec, bspec, wspec, bspec, wspec2, bspec,
        ],
        out_specs=pl.BlockSpec((BS, D), lambda i: (i, 0)),
        out_shape=jax.ShapeDtypeStruct((B, D), jnp.float32),
    )(input_val, W0a, b0a, W0b, b0b, W1p, b1p, W1q, b1q)
    return out


# speedup vs baseline: 1.0856x; 1.0856x over previous
import jax
import jax.numpy as jnp
from jax.experimental import pallas as pl

B = 16384
D = 128
BS = 4096

_DNT = (((1,), (1,)), ((), ()))


def _fused_mlp(xin_ref, w0a_ref, b0a_ref, w0b_ref, b0b_ref, w1p_ref, b1p_ref,
               w1q_ref, b1q_ref, o_ref):
    f32 = jnp.float32
    x0 = xin_ref[:, :D]
    x1 = xin_ref[:, D:2 * D]
    w0a, w0b, w1p, w1q = w0a_ref[0], w0b_ref[0], w1p_ref[0], w1q_ref[0]
    h0 = jnp.maximum(
        jax.lax.dot_general(x0, w0a, _DNT, preferred_element_type=f32)
        + b0a_ref[0:1, :], 0.0)
    x = jnp.maximum(
        jax.lax.dot_general(h0, w0b, _DNT, preferred_element_type=f32)
        + b0b_ref[0:1, :], 0.0)
    h1 = jnp.maximum(
        jax.lax.dot_general(x1, w1p, _DNT, preferred_element_type=f32)
        + b1p_ref[0:1, :], 0.0)
    o_ref[...] = (
        jax.lax.dot_general(x, w1q[:, :D], _DNT, preferred_element_type=f32)
        + jax.lax.dot_general(h1, w1q[:, D:], _DNT, preferred_element_type=f32)
        + b1q_ref[0:1, :])


def kernel(input_val, W0a, b0a, W0b, b0b, W1p, b1p, W1q, b1q):
    n_blocks = B // BS
    wspec = pl.BlockSpec((1, D, D), lambda i: (0, 0, 0))
    wspec2 = pl.BlockSpec((1, D, 2 * D), lambda i: (0, 0, 0))
    bspec = pl.BlockSpec((8, D), lambda i: (0, 0))
    out = pl.pallas_call(
        _fused_mlp,
        grid=(n_blocks,),
        in_specs=[
            pl.BlockSpec((BS, 2 * D), lambda i: (i, 0)),
            wspec, bspec, wspec, bspec, wspec, bspec, wspec2, bspec,
        ],
        out_specs=pl.BlockSpec((BS, D), lambda i: (i, 0)),
        out_shape=jax.ShapeDtypeStruct((B, D), jnp.float32),
    )(input_val, W0a, b0a, W0b, b0b, W1p, b1p, W1q, b1q)
    return out
